# baseline (speedup 1.0000x reference)
import functools

import jax
import jax.numpy as jnp
from jax import lax
from jax.experimental import pallas as pl
from jax.experimental.pallas import tpu as pltpu
from jax.experimental.pallas import tpu_sc as plsc

B = 1024
VOCAB = 100000
D = 128
N_TOTAL = B + B * 25 + B * 250

NW = 32
G1 = B * 25
PS = G1 // NW
SUB = 80
NSUB = PS // SUB
NPASS = 10
H0_ROWS_W = B // NW


def _side_gather(wid, gn_hbm, emb_hbm, nb0_hbm, nb1_hbm, nb2p_hbm,
                 h0_out, h1_out, m2_out,
                 out_v, q_v, ids_v, nb0_v, ids0_v,
                 sem_q, sem_id, sem_add, sem_out, sem):
  g0 = wid * PS

  def fire_q(nb_hbm, r, b):
    pltpu.async_copy(nb_hbm.at[pl.ds(r * G1 + g0, PS)], q_v[b], sem_q.at[b])

  def wait_q(b):
    pltpu.make_async_copy(nb2p_hbm.at[pl.ds(0, PS)], q_v[b],
                          sem_q.at[b]).wait()

  def fire_idg(b):
    for j in range(NSUB):
      pltpu.async_copy(gn_hbm.at[q_v[b].at[pl.ds(j * SUB, SUB)]],
                       ids_v[b].at[pl.ds(j * SUB, SUB)], sem_id.at[b])

  def wait_idg(b):
    pltpu.make_async_copy(nb2p_hbm.at[pl.ds(0, PS)], ids_v[b],
                          sem_id.at[b]).wait()

  def fire_rows(b, add):
    for j in range(NSUB):
      pltpu.async_copy(emb_hbm.at[ids_v[b].at[pl.ds(j * SUB, SUB)]],
                       out_v.at[pl.ds(j * SUB, SUB), :], sem_add, add=add)

  def wait_rows():
    pltpu.make_async_copy(emb_hbm.at[pl.ds(0, PS), :], out_v, sem_add).wait()

  fire_q(nb2p_hbm, 0, 0)
  wait_q(0)
  fire_idg(0)
  fire_q(nb2p_hbm, 1, 1)
  wait_idg(0)
  fire_rows(0, False)
  wait_q(1)
  fire_idg(1)
  fire_q(nb2p_hbm, 2, 0)

  def pair_body(p, carry):
    for q in range(2):
      r = 2 * p + 1 + q
      b = 1 - q
      wait_idg(b)
      wait_rows()
      fire_rows(b, True)
      wait_q(1 - b)
      fire_idg(1 - b)
      fire_q(nb2p_hbm, r + 2, b)
    return carry

  lax.fori_loop(0, (NPASS - 4) // 2, pair_body, 0)

  for r in (NPASS - 3, NPASS - 2, NPASS - 1):
    b = r % 2
    wait_idg(b)
    wait_rows()
    fire_rows(b, True)
    if r < NPASS - 1:
      wait_q(1 - b)
      fire_idg(1 - b)
    if r < NPASS - 2:
      fire_q(nb2p_hbm, r + 2, b)
  wait_rows()
  pltpu.async_copy(out_v, m2_out.at[pl.ds(g0, PS), :], sem_out)

  pltpu.async_copy(nb1_hbm.at[pl.ds(g0, PS)], q_v[0], sem_q.at[0])
  wait_q(0)
  fire_idg(0)
  wait_idg(0)
  pltpu.make_async_copy(out_v, m2_out.at[pl.ds(0, PS), :],
                        sem_out).wait()
  fire_rows(0, False)
  wait_rows()
  pltpu.async_copy(out_v, h1_out.at[pl.ds(g0, PS), :], sem_out)

  pltpu.sync_copy(nb0_hbm.at[pl.ds(wid * H0_ROWS_W, H0_ROWS_W)], nb0_v)
  pltpu.async_copy(gn_hbm.at[nb0_v], ids0_v, sem).wait()
  pltpu.make_async_copy(out_v, h1_out.at[pl.ds(0, PS), :],
                        sem_out).wait()
  pltpu.async_copy(emb_hbm.at[ids0_v],
                   out_v.at[pl.ds(0, H0_ROWS_W), :], sem).wait()
  pltpu.sync_copy(out_v.at[pl.ds(0, H0_ROWS_W), :],
                  h0_out.at[pl.ds(wid * H0_ROWS_W, H0_ROWS_W), :])


def _sc_body(src_gn, dst_gn, emb,
             snb0, snb1, snb2p, dnb0, dnb1, dnb2p,
             s_h0, s_h1, s_m2, d_h0, d_h1, d_m2,
             out_v, q_v0, q_v1, ids_v0, ids_v1, nb0_v, ids0_v,
             sem_q, sem_id, sem_add, sem_out, sem):
  wid = lax.axis_index("s") * 2 + lax.axis_index("c")
  for (gn, nb0, nb1, nb2p, h0, h1, m2) in (
      (src_gn, snb0, snb1, snb2p, s_h0, s_h1, s_m2),
      (dst_gn, dnb0, dnb1, dnb2p, d_h0, d_h1, d_m2)):
    _side_gather(wid, gn, emb, nb0, nb1, nb2p, h0, h1, m2,
                 out_v, (q_v0, q_v1), (ids_v0, ids_v1), nb0_v, ids0_v,
                 sem_q, sem_id, sem_add, sem_out, sem)


_f32 = jnp.float32
_sc_gather = pl.kernel(
    _sc_body,
    out_type=[
        jax.ShapeDtypeStruct((B, D), _f32),
        jax.ShapeDtypeStruct((G1, D), _f32),
        jax.ShapeDtypeStruct((G1, D), _f32),
        jax.ShapeDtypeStruct((B, D), _f32),
        jax.ShapeDtypeStruct((G1, D), _f32),
        jax.ShapeDtypeStruct((G1, D), _f32),
    ],
    mesh=plsc.VectorSubcoreMesh(core_axis_name="c", subcore_axis_name="s"),
    scratch_types=[
        pltpu.VMEM((PS, D), _f32),
        pltpu.VMEM((PS,), jnp.int32),
        pltpu.VMEM((PS,), jnp.int32),
        pltpu.VMEM((PS,), jnp.int32),
        pltpu.VMEM((PS,), jnp.int32),
        pltpu.VMEM((H0_ROWS_W,), jnp.int32),
        pltpu.VMEM((H0_ROWS_W,), jnp.int32),
        pltpu.SemaphoreType.DMA((2,)),
        pltpu.SemaphoreType.DMA((2,)),
        pltpu.SemaphoreType.DMA,
        pltpu.SemaphoreType.DMA,
        pltpu.SemaphoreType.DMA,
    ],
)


BLK = 1600
GRID = G1 // BLK
GRP_BLK = BLK // 25


def _tc_body(h1s, m2s, h1d, m2d, h0s, h0d, lab, agg,
             W0s, b0s, W1s, b1s, W0d, b0d, W1d, b1d,
             preds, loss, m1s, mh1s, m1d, mh1d):
  i = pl.program_id(0)
  ag = agg[...]

  for (h1, m2, W0, b0, m1, mh1) in (
      (h1s, m2s, W0s, b0s, m1s, mh1s),
      (h1d, m2d, W0d, b0d, m1d, mh1d)):
    sv = h1[...]
    nm = m2[...] * 0.1
    x = jnp.concatenate([sv, nm], axis=1)
    out1 = jnp.maximum(jnp.dot(x, W0[...],
                               preferred_element_type=_f32) + b0[...], 0.0)
    m1[pl.ds(i * GRP_BLK, GRP_BLK), :] = jnp.dot(
        ag, out1, preferred_element_type=_f32)
    mh1[pl.ds(i * GRP_BLK, GRP_BLK), :] = jnp.dot(
        ag, sv, preferred_element_type=_f32)

  @pl.when(i == GRID - 1)
  def _final():
    outs = []
    for (h0, W0, b0, W1, b1, m1, mh1) in (
        (h0s, W0s, b0s, W1s, b1s, m1s, mh1s),
        (h0d, W0d, b0d, W1d, b1d, m1d, mh1d)):
      x0 = jnp.concatenate([h0[...], mh1[...]], axis=1)
      out0 = jnp.maximum(jnp.dot(x0, W0[...],
                                 preferred_element_type=_f32) + b0[...], 0.0)
      xf = jnp.concatenate([out0, m1[...]], axis=1)
      fin = jnp.dot(xf, W1[...], preferred_element_type=_f32) + b1[...]
      ss = jnp.sum(fin * fin, axis=1, keepdims=True)
      outs.append(fin * lax.rsqrt(jnp.maximum(ss, 1e-12)))
    logits = jnp.sum(outs[0] * outs[1], axis=1, keepdims=True)
    preds[...] = jax.nn.sigmoid(logits)
    lv = lab[...]
    ent = (jnp.maximum(logits, 0.0) - logits * lv
           + jnp.log1p(jnp.exp(-jnp.abs(logits))))
    loss[...] = jnp.mean(ent).reshape(1, 1)


def _tc_dense(h1s, m2s, h1d, m2d, h0s, h0d, lab, agg,
              W0s, b0s, W1s, b1s, W0d, b0d, W1d, b1d):
  blk = pl.BlockSpec((BLK, D), lambda i: (i, 0))
  full = lambda shape: pl.BlockSpec(shape, lambda i: (0, 0))
  return pl.pallas_call(
      _tc_body,
      grid=(GRID,),
      in_specs=[
          blk, blk, blk, blk,
          full((B, D)), full((B, D)), full((B, 1)), full((GRP_BLK, BLK)),
          full((2 * D, 2 * D)), full((1, 2 * D)),
          full((4 * D, 2 * D)), full((1, 2 * D)),
          full((2 * D, 2 * D)), full((1, 2 * D)),
          full((4 * D, 2 * D)), full((1, 2 * D)),
      ],
      out_specs=[full((B, 1)), full((1, 1))],
      out_shape=[
          jax.ShapeDtypeStruct((B, 1), _f32),
          jax.ShapeDtypeStruct((1, 1), _f32),
      ],
      scratch_shapes=[
          pltpu.VMEM((B, 2 * D), _f32),
          pltpu.VMEM((B, D), _f32),
          pltpu.VMEM((B, 2 * D), _f32),
          pltpu.VMEM((B, D), _f32),
      ],
  )(h1s, m2s, h1d, m2d, h0s, h0d, lab, agg,
    W0s, b0s, W1s, b1s, W0d, b0d, W1d, b1d)


def _blockdiag(a, b):
  da, n = a.shape
  db, _ = b.shape
  z = jnp.zeros((da + db, 2 * n), _f32)
  z = z.at[:da, :n].set(a)
  return z.at[da:, n:].set(b)


def kernel(src_graph_nodes, dst_graph_nodes, labels,
           src_nb0, src_nb1, src_nb2, dst_nb0, dst_nb1, dst_nb2,
           node_emb,
           src_Wself0, src_Wneigh0, src_b0, src_Wself1, src_Wneigh1, src_b1,
           dst_Wself0, dst_Wneigh0, dst_b0, dst_Wself1, dst_Wneigh1, dst_b1):
  i32 = jnp.int32
  sgn = src_graph_nodes.astype(i32)
  dgn = dst_graph_nodes.astype(i32)
  snb0 = src_nb0.astype(i32)
  snb1 = src_nb1.astype(i32)
  dnb0 = dst_nb0.astype(i32)
  dnb1 = dst_nb1.astype(i32)
  snb2p = src_nb2.astype(i32).reshape(G1, NPASS).T.reshape(-1)
  dnb2p = dst_nb2.astype(i32).reshape(G1, NPASS).T.reshape(-1)

  s_h0, s_h1, s_m2, d_h0, d_h1, d_m2 = _sc_gather(
      sgn, dgn, node_emb, snb0, snb1, snb2p, dnb0, dnb1, dnb2p)

  W0s = _blockdiag(src_Wself0, src_Wneigh0)
  W1s = _blockdiag(src_Wself1, src_Wneigh1)
  W0d = _blockdiag(dst_Wself0, dst_Wneigh0)
  W1d = _blockdiag(dst_Wself1, dst_Wneigh1)

  agg = jnp.where(
      (jax.lax.broadcasted_iota(jnp.int32, (GRP_BLK, BLK), 1) // 25)
      == jax.lax.broadcasted_iota(jnp.int32, (GRP_BLK, BLK), 0),
      1.0 / 25.0, 0.0).astype(_f32)

  preds, loss = _tc_dense(
      s_h1, s_m2, d_h1, d_m2, s_h0, d_h0, labels.reshape(B, 1), agg,
      W0s, src_b0.reshape(1, 2 * D), W1s, src_b1.reshape(1, 2 * D),
      W0d, dst_b0.reshape(1, 2 * D), W1d, dst_b1.reshape(1, 2 * D))
  return preds, loss[0, 0]


# speedup vs baseline: 1.1857x; 1.1857x over previous
import functools

import jax
import jax.numpy as jnp
from jax import lax
from jax.experimental import pallas as pl
from jax.experimental.pallas import tpu as pltpu
from jax.experimental.pallas import tpu_sc as plsc

B = 1024
VOCAB = 100000
D = 128
N_TOTAL = B + B * 25 + B * 250

NW = 32
SUB = 80
CH = 400
NSUB = CH // SUB

H2_ROWS_W = 256000 // NW
H2_CHUNKS = H2_ROWS_W // CH
G_CH = CH // 10
H1_ROWS_W = 25600 // NW
H1_CHUNKS = H1_ROWS_W // CH
H0_ROWS_W = B // NW


def _side_gather(wid, gn_hbm, emb_hbm, nb0_hbm, nb1_hbm, nb2_hbm,
                 h0_out, h1_out, m2_out,
                 nb_v, ids_v, rows_v, out_v, nb0_v, ids0_v,
                 sem_nb, sem_id, sem_row, sem_out, sem):
  def fire_nb(nb_hbm, start, c, b):
    pltpu.async_copy(nb_hbm.at[pl.ds(start + c * CH, CH)], nb_v[b],
                     sem_nb.at[b])

  def wait_nb(b):
    pltpu.make_async_copy(nb2_hbm.at[pl.ds(0, CH)], nb_v[b],
                          sem_nb.at[b]).wait()

  def fire_ids(b):
    for j in range(NSUB):
      pltpu.async_copy(gn_hbm.at[nb_v[b].at[pl.ds(j * SUB, SUB)]],
                       ids_v[b].at[pl.ds(j * SUB, SUB)], sem_id.at[b])

  def wait_ids(b):
    pltpu.make_async_copy(nb2_hbm.at[pl.ds(0, CH)], ids_v[b],
                          sem_id.at[b]).wait()

  def fire_rows(b):
    for j in range(NSUB):
      pltpu.async_copy(emb_hbm.at[ids_v[b].at[pl.ds(j * SUB, SUB)]],
                       rows_v[b].at[pl.ds(j * SUB, SUB), :], sem_row.at[b])

  def wait_rows(b):
    pltpu.make_async_copy(emb_hbm.at[pl.ds(0, CH), :], rows_v[b],
                          sem_row.at[b]).wait()

  m2_base = wid * (H2_ROWS_W // 10)
  h2_start = wid * H2_ROWS_W

  def fire_out(c, b):
    pltpu.async_copy(out_v[b], m2_out.at[pl.ds(m2_base + c * G_CH, G_CH), :],
                     sem_out.at[b])

  def wait_out(b):
    pltpu.make_async_copy(out_v[b], m2_out.at[pl.ds(0, G_CH), :],
                          sem_out.at[b]).wait()

  def sum_chunk(b, unroll=1):
    @plsc.parallel_loop(0, G_CH, unroll=unroll)
    def g_body(g):
      for j in range(D // 16):
        acc = rows_v[b][10 * g, pl.ds(j * 16, 16)]
        for r in range(1, 10):
          acc = acc + rows_v[b][10 * g + r, pl.ds(j * 16, 16)]
        out_v[b][g, pl.ds(j * 16, 16)] = acc

  def h2_step(c, b, n_live):
    if n_live >= 1:
      wait_ids(1 - b)
      fire_rows(1 - b)
    wait_rows(b)
    if n_live >= 2:
      wait_nb(b)
      fire_ids(b)
    if n_live >= 3:
      fire_nb(nb2_hbm, h2_start, c + 3, 1 - b)

  fire_nb(nb2_hbm, h2_start, 0, 0)
  wait_nb(0)
  fire_ids(0)
  fire_nb(nb2_hbm, h2_start, 1, 1)
  wait_ids(0)
  fire_rows(0)
  wait_nb(1)
  fire_ids(1)
  fire_nb(nb2_hbm, h2_start, 2, 0)
  fire_out(0, 0)
  fire_out(1, 1)

  def pair_body(p, carry):
    for q in range(2):
      c = 2 * p + q
      h2_step(c, q, 3)
      wait_out(q)
      sum_chunk(q, unroll=2)
      fire_out(c, q)
    return carry

  lax.fori_loop(0, H2_CHUNKS // 2 - 2, pair_body, 0)

  for c in range(H2_CHUNKS - 4, H2_CHUNKS):
    b = c % 2
    h2_step(c, b, H2_CHUNKS - 1 - c)
    wait_out(b)
    sum_chunk(b)
    fire_out(c, b)
  wait_out(0)
  wait_out(1)

  h1_start = wid * H1_ROWS_W
  fire_nb(nb1_hbm, h1_start, 0, 0)
  wait_nb(0)
  fire_ids(0)
  fire_nb(nb1_hbm, h1_start, 1, 1)
  wait_ids(0)
  fire_rows(0)
  wait_nb(1)
  fire_ids(1)
  wait_ids(1)
  fire_rows(1)
  wait_rows(0)
  pltpu.async_copy(rows_v[0], h1_out.at[pl.ds(h1_start, CH), :],
                   sem_out.at[0])
  wait_rows(1)
  pltpu.async_copy(rows_v[1], h1_out.at[pl.ds(h1_start + CH, CH), :],
                   sem_out.at[1])
  pltpu.make_async_copy(rows_v[0], h1_out.at[pl.ds(0, CH), :],
                        sem_out.at[0]).wait()
  pltpu.make_async_copy(rows_v[1], h1_out.at[pl.ds(0, CH), :],
                        sem_out.at[1]).wait()

  pltpu.sync_copy(nb0_hbm.at[pl.ds(wid * H0_ROWS_W, H0_ROWS_W)], nb0_v)
  pltpu.async_copy(gn_hbm.at[nb0_v], ids0_v, sem).wait()
  pltpu.async_copy(emb_hbm.at[ids0_v],
                   rows_v[0].at[pl.ds(0, H0_ROWS_W), :], sem).wait()
  pltpu.sync_copy(rows_v[0].at[pl.ds(0, H0_ROWS_W), :],
                  h0_out.at[pl.ds(wid * H0_ROWS_W, H0_ROWS_W), :])


def _sc_body(src_gn, dst_gn, emb,
             snb0, snb1, snb2, dnb0, dnb1, dnb2,
             s_h0, s_h1, s_m2, d_h0, d_h1, d_m2,
             nb_v0, nb_v1, ids_v0, ids_v1, rows_v0, rows_v1,
             out_v0, out_v1, nb0_v, ids0_v,
             sem_nb, sem_id, sem_row, sem_out, sem):
  wid = lax.axis_index("s") * 2 + lax.axis_index("c")
  for (gn, nb0, nb1, nb2, h0, h1, m2) in (
      (src_gn, snb0, snb1, snb2, s_h0, s_h1, s_m2),
      (dst_gn, dnb0, dnb1, dnb2, d_h0, d_h1, d_m2)):
    _side_gather(wid, gn, emb, nb0, nb1, nb2, h0, h1, m2,
                 (nb_v0, nb_v1), (ids_v0, ids_v1), (rows_v0, rows_v1),
                 (out_v0, out_v1), nb0_v, ids0_v,
                 sem_nb, sem_id, sem_row, sem_out, sem)


_f32 = jnp.float32
_sc_gather = pl.kernel(
    _sc_body,
    out_type=[
        jax.ShapeDtypeStruct((B, D), _f32),
        jax.ShapeDtypeStruct((25600, D), _f32),
        jax.ShapeDtypeStruct((25600, D), _f32),
        jax.ShapeDtypeStruct((B, D), _f32),
        jax.ShapeDtypeStruct((25600, D), _f32),
        jax.ShapeDtypeStruct((25600, D), _f32),
    ],
    mesh=plsc.VectorSubcoreMesh(core_axis_name="c", subcore_axis_name="s"),
    scratch_types=[
        pltpu.VMEM((CH,), jnp.int32),
        pltpu.VMEM((CH,), jnp.int32),
        pltpu.VMEM((CH,), jnp.int32),
        pltpu.VMEM((CH,), jnp.int32),
        pltpu.VMEM((CH, D), _f32),
        pltpu.VMEM((CH, D), _f32),
        pltpu.VMEM((G_CH, D), _f32),
        pltpu.VMEM((G_CH, D), _f32),
        pltpu.VMEM((H0_ROWS_W,), jnp.int32),
        pltpu.VMEM((H0_ROWS_W,), jnp.int32),
        pltpu.SemaphoreType.DMA((2,)),
        pltpu.SemaphoreType.DMA((2,)),
        pltpu.SemaphoreType.DMA((2,)),
        pltpu.SemaphoreType.DMA((2,)),
        pltpu.SemaphoreType.DMA,
    ],
)


BLK = 1600
GRID = 25600 // BLK
GRP_BLK = BLK // 25


def _tc_body(h1s, m2s, h1d, m2d, h0s, h0d, lab, agg,
             W0s, b0s, W1s, b1s, W0d, b0d, W1d, b1d,
             preds, loss, m1s, mh1s, m1d, mh1d):
  i = pl.program_id(0)
  ag = agg[...]

  for (h1, m2, W0, b0, m1, mh1) in (
      (h1s, m2s, W0s, b0s, m1s, mh1s),
      (h1d, m2d, W0d, b0d, m1d, mh1d)):
    sv = h1[...]
    nm = m2[...] * 0.1
    x = jnp.concatenate([sv, nm], axis=1)
    out1 = jnp.maximum(jnp.dot(x, W0[...],
                               preferred_element_type=_f32) + b0[...], 0.0)
    m1[pl.ds(i * GRP_BLK, GRP_BLK), :] = jnp.dot(
        ag, out1, preferred_element_type=_f32)
    mh1[pl.ds(i * GRP_BLK, GRP_BLK), :] = jnp.dot(
        ag, sv, preferred_element_type=_f32)

  @pl.when(i == GRID - 1)
  def _final():
    outs = []
    for (h0, W0, b0, W1, b1, m1, mh1) in (
        (h0s, W0s, b0s, W1s, b1s, m1s, mh1s),
        (h0d, W0d, b0d, W1d, b1d, m1d, mh1d)):
      x0 = jnp.concatenate([h0[...], mh1[...]], axis=1)
      out0 = jnp.maximum(jnp.dot(x0, W0[...],
                                 preferred_element_type=_f32) + b0[...], 0.0)
      xf = jnp.concatenate([out0, m1[...]], axis=1)
      fin = jnp.dot(xf, W1[...], preferred_element_type=_f32) + b1[...]
      ss = jnp.sum(fin * fin, axis=1, keepdims=True)
      outs.append(fin * lax.rsqrt(jnp.maximum(ss, 1e-12)))
    logits = jnp.sum(outs[0] * outs[1], axis=1, keepdims=True)
    preds[...] = jax.nn.sigmoid(logits)
    lv = lab[...]
    ent = (jnp.maximum(logits, 0.0) - logits * lv
           + jnp.log1p(jnp.exp(-jnp.abs(logits))))
    loss[...] = jnp.mean(ent).reshape(1, 1)


def _tc_dense(h1s, m2s, h1d, m2d, h0s, h0d, lab, agg,
              W0s, b0s, W1s, b1s, W0d, b0d, W1d, b1d):
  blk = pl.BlockSpec((BLK, D), lambda i: (i, 0))
  full = lambda shape: pl.BlockSpec(shape, lambda i: (0, 0))
  return pl.pallas_call(
      _tc_body,
      grid=(GRID,),
      in_specs=[
          blk, blk, blk, blk,
          full((B, D)), full((B, D)), full((B, 1)), full((GRP_BLK, BLK)),
          full((2 * D, 2 * D)), full((1, 2 * D)),
          full((4 * D, 2 * D)), full((1, 2 * D)),
          full((2 * D, 2 * D)), full((1, 2 * D)),
          full((4 * D, 2 * D)), full((1, 2 * D)),
      ],
      out_specs=[full((B, 1)), full((1, 1))],
      out_shape=[
          jax.ShapeDtypeStruct((B, 1), _f32),
          jax.ShapeDtypeStruct((1, 1), _f32),
      ],
      scratch_shapes=[
          pltpu.VMEM((B, 2 * D), _f32),
          pltpu.VMEM((B, D), _f32),
          pltpu.VMEM((B, 2 * D), _f32),
          pltpu.VMEM((B, D), _f32),
      ],
  )(h1s, m2s, h1d, m2d, h0s, h0d, lab, agg,
    W0s, b0s, W1s, b1s, W0d, b0d, W1d, b1d)


def _blockdiag(a, b):
  da, n = a.shape
  db, _ = b.shape
  z = jnp.zeros((da + db, 2 * n), _f32)
  z = z.at[:da, :n].set(a)
  return z.at[da:, n:].set(b)


def kernel(src_graph_nodes, dst_graph_nodes, labels,
           src_nb0, src_nb1, src_nb2, dst_nb0, dst_nb1, dst_nb2,
           node_emb,
           src_Wself0, src_Wneigh0, src_b0, src_Wself1, src_Wneigh1, src_b1,
           dst_Wself0, dst_Wneigh0, dst_b0, dst_Wself1, dst_Wneigh1, dst_b1):
  i32 = jnp.int32
  sgn = src_graph_nodes.astype(i32)
  dgn = dst_graph_nodes.astype(i32)
  snb0 = src_nb0.astype(i32)
  snb1 = src_nb1.astype(i32)
  snb2 = src_nb2.astype(i32)
  dnb0 = dst_nb0.astype(i32)
  dnb1 = dst_nb1.astype(i32)
  dnb2 = dst_nb2.astype(i32)

  s_h0, s_h1, s_m2, d_h0, d_h1, d_m2 = _sc_gather(
      sgn, dgn, node_emb, snb0, snb1, snb2, dnb0, dnb1, dnb2)

  W0s = _blockdiag(src_Wself0, src_Wneigh0)
  W1s = _blockdiag(src_Wself1, src_Wneigh1)
  W0d = _blockdiag(dst_Wself0, dst_Wneigh0)
  W1d = _blockdiag(dst_Wself1, dst_Wneigh1)

  agg = jnp.where(
      (jax.lax.broadcasted_iota(jnp.int32, (GRP_BLK, BLK), 1) // 25)
      == jax.lax.broadcasted_iota(jnp.int32, (GRP_BLK, BLK), 0),
      1.0 / 25.0, 0.0).astype(_f32)

  preds, loss = _tc_dense(
      s_h1, s_m2, d_h1, d_m2, s_h0, d_h0, labels.reshape(B, 1), agg,
      W0s, src_b0.reshape(1, 2 * D), W1s, src_b1.reshape(1, 2 * D),
      W0d, dst_b0.reshape(1, 2 * D), W1d, dst_b1.reshape(1, 2 * D))
  return preds, loss[0, 0]


# speedup vs baseline: 1.2019x; 1.0137x over previous
import functools

import jax
import jax.numpy as jnp
from jax import lax
from jax.experimental import pallas as pl
from jax.experimental.pallas import tpu as pltpu
from jax.experimental.pallas import tpu_sc as plsc

B = 1024
VOCAB = 100000
D = 128
N_TOTAL = B + B * 25 + B * 250

NW = 32
SUB = 80
CH = 400
NSUB = CH // SUB

H2_ROWS_W = 256000 // NW
H2_CHUNKS = H2_ROWS_W // CH
G_CH = CH // 10
H1_ROWS_W = 25600 // NW
H1_CHUNKS = H1_ROWS_W // CH
H0_ROWS_W = B // NW


def _side_gather(wid, gn_hbm, emb_hbm, nb0_hbm, nb1_hbm, nb2_hbm,
                 h0_out, h1_out, m2_out,
                 nb_v, ids_v, rows_v, out_v, nb0_v, ids0_v,
                 sem_nb, sem_id, sem_row, sem_out, sem):
  def fire_nb(nb_hbm, start, c, b):
    pltpu.async_copy(nb_hbm.at[pl.ds(start + c * CH, CH)], nb_v[b],
                     sem_nb.at[b])

  def wait_nb(b):
    pltpu.make_async_copy(nb2_hbm.at[pl.ds(0, CH)], nb_v[b],
                          sem_nb.at[b]).wait()

  def fire_ids(b):
    for j in range(NSUB):
      pltpu.async_copy(gn_hbm.at[nb_v[b].at[pl.ds(j * SUB, SUB)]],
                       ids_v[b].at[pl.ds(j * SUB, SUB)], sem_id.at[b])

  def wait_ids(b):
    pltpu.make_async_copy(nb2_hbm.at[pl.ds(0, CH)], ids_v[b],
                          sem_id.at[b]).wait()

  def fire_rows(b):
    for j in range(NSUB):
      pltpu.async_copy(emb_hbm.at[ids_v[b].at[pl.ds(j * SUB, SUB)]],
                       rows_v[b].at[pl.ds(j * SUB, SUB), :], sem_row.at[b])

  def wait_rows(b):
    pltpu.make_async_copy(emb_hbm.at[pl.ds(0, CH), :], rows_v[b],
                          sem_row.at[b]).wait()

  m2_base = wid * (H2_ROWS_W // 10)
  h2_start = wid * H2_ROWS_W

  def fire_out(c, b):
    pltpu.async_copy(out_v[b], m2_out.at[pl.ds(m2_base + c * G_CH, G_CH), :],
                     sem_out.at[b])

  def wait_out(b):
    pltpu.make_async_copy(out_v[b], m2_out.at[pl.ds(0, G_CH), :],
                          sem_out.at[b]).wait()

  def sum_chunk(b, unroll=1):
    @plsc.parallel_loop(0, G_CH, unroll=unroll)
    def g_body(g):
      for j in range(D // 16):
        acc = rows_v[b][10 * g, pl.ds(j * 16, 16)]
        for r in range(1, 10):
          acc = acc + rows_v[b][10 * g + r, pl.ds(j * 16, 16)]
        out_v[b][g, pl.ds(j * 16, 16)] = acc

  def h2_step(c, b, n_live):
    if n_live >= 1:
      wait_ids(1 - b)
      fire_rows(1 - b)
    wait_rows(b)
    if n_live >= 2:
      wait_nb(b)
      fire_ids(b)
    if n_live >= 3:
      fire_nb(nb2_hbm, h2_start, c + 3, 1 - b)

  fire_nb(nb2_hbm, h2_start, 0, 0)
  wait_nb(0)
  fire_ids(0)
  fire_nb(nb2_hbm, h2_start, 1, 1)
  wait_ids(0)
  fire_rows(0)
  wait_nb(1)
  fire_ids(1)
  fire_nb(nb2_hbm, h2_start, 2, 0)
  fire_out(0, 0)
  fire_out(1, 1)

  def pair_body(p, carry):
    for q in range(2):
      c = 2 * p + q
      h2_step(c, q, 3)
      wait_out(q)
      sum_chunk(q, unroll=2)
      fire_out(c, q)
    return carry

  lax.fori_loop(0, H2_CHUNKS // 2 - 2, pair_body, 0)

  for c in range(H2_CHUNKS - 4, H2_CHUNKS):
    b = c % 2
    h2_step(c, b, H2_CHUNKS - 1 - c)
    wait_out(b)
    sum_chunk(b)
    fire_out(c, b)
  wait_out(0)
  wait_out(1)

  h1_start = wid * H1_ROWS_W
  fire_nb(nb1_hbm, h1_start, 0, 0)
  wait_nb(0)
  fire_ids(0)
  fire_nb(nb1_hbm, h1_start, 1, 1)
  wait_ids(0)
  fire_rows(0)
  wait_nb(1)
  fire_ids(1)
  wait_ids(1)
  fire_rows(1)
  wait_rows(0)
  pltpu.async_copy(rows_v[0], h1_out.at[pl.ds(h1_start, CH), :],
                   sem_out.at[0])
  wait_rows(1)
  pltpu.async_copy(rows_v[1], h1_out.at[pl.ds(h1_start + CH, CH), :],
                   sem_out.at[1])
  pltpu.make_async_copy(rows_v[0], h1_out.at[pl.ds(0, CH), :],
                        sem_out.at[0]).wait()
  pltpu.make_async_copy(rows_v[1], h1_out.at[pl.ds(0, CH), :],
                        sem_out.at[1]).wait()

  pltpu.sync_copy(nb0_hbm.at[pl.ds(wid * H0_ROWS_W, H0_ROWS_W)], nb0_v)
  pltpu.async_copy(gn_hbm.at[nb0_v], ids0_v, sem).wait()
  pltpu.async_copy(emb_hbm.at[ids0_v],
                   rows_v[0].at[pl.ds(0, H0_ROWS_W), :], sem).wait()
  pltpu.sync_copy(rows_v[0].at[pl.ds(0, H0_ROWS_W), :],
                  h0_out.at[pl.ds(wid * H0_ROWS_W, H0_ROWS_W), :])


def _sc_body(src_gn, dst_gn, emb,
             snb0, snb1, snb2, dnb0, dnb1, dnb2,
             s_h0, s_h1, s_m2, d_h0, d_h1, d_m2,
             nb_v0, nb_v1, ids_v0, ids_v1, rows_v0, rows_v1,
             out_v0, out_v1, nb0_v, ids0_v,
             sem_nb, sem_id, sem_row, sem_out, sem):
  wid = lax.axis_index("s") * 2 + lax.axis_index("c")
  for (gn, nb0, nb1, nb2, h0, h1, m2) in (
      (src_gn, snb0, snb1, snb2, s_h0, s_h1, s_m2),
      (dst_gn, dnb0, dnb1, dnb2, d_h0, d_h1, d_m2)):
    _side_gather(wid, gn, emb, nb0, nb1, nb2, h0, h1, m2,
                 (nb_v0, nb_v1), (ids_v0, ids_v1), (rows_v0, rows_v1),
                 (out_v0, out_v1), nb0_v, ids0_v,
                 sem_nb, sem_id, sem_row, sem_out, sem)


_f32 = jnp.float32
_sc_gather = pl.kernel(
    _sc_body,
    out_type=[
        jax.ShapeDtypeStruct((B, D), _f32),
        jax.ShapeDtypeStruct((25600, D), _f32),
        jax.ShapeDtypeStruct((25600, D), _f32),
        jax.ShapeDtypeStruct((B, D), _f32),
        jax.ShapeDtypeStruct((25600, D), _f32),
        jax.ShapeDtypeStruct((25600, D), _f32),
    ],
    mesh=plsc.VectorSubcoreMesh(core_axis_name="c", subcore_axis_name="s"),
    scratch_types=[
        pltpu.VMEM((CH,), jnp.int32),
        pltpu.VMEM((CH,), jnp.int32),
        pltpu.VMEM((CH,), jnp.int32),
        pltpu.VMEM((CH,), jnp.int32),
        pltpu.VMEM((CH, D), _f32),
        pltpu.VMEM((CH, D), _f32),
        pltpu.VMEM((G_CH, D), _f32),
        pltpu.VMEM((G_CH, D), _f32),
        pltpu.VMEM((H0_ROWS_W,), jnp.int32),
        pltpu.VMEM((H0_ROWS_W,), jnp.int32),
        pltpu.SemaphoreType.DMA((2,)),
        pltpu.SemaphoreType.DMA((2,)),
        pltpu.SemaphoreType.DMA((2,)),
        pltpu.SemaphoreType.DMA((2,)),
        pltpu.SemaphoreType.DMA,
    ],
)


BLK = 3200
GRID = 25600 // BLK
GRP_BLK = BLK // 25


def _tc_body(h1s, m2s, h1d, m2d, h0s, h0d, lab, agg,
             W0s, b0s, W1s, b1s, W0d, b0d, W1d, b1d,
             preds, loss, m1s, mh1s, m1d, mh1d):
  i = pl.program_id(0)
  ag = agg[...]

  for (h1, m2, W0, b0, m1, mh1) in (
      (h1s, m2s, W0s, b0s, m1s, mh1s),
      (h1d, m2d, W0d, b0d, m1d, mh1d)):
    sv = h1[...]
    nm = m2[...] * 0.1
    x = jnp.concatenate([sv, nm], axis=1)
    out1 = jnp.maximum(jnp.dot(x, W0[...],
                               preferred_element_type=_f32) + b0[...], 0.0)
    m1[pl.ds(i * GRP_BLK, GRP_BLK), :] = jnp.dot(
        ag, out1, preferred_element_type=_f32)
    mh1[pl.ds(i * GRP_BLK, GRP_BLK), :] = jnp.dot(
        ag, sv, preferred_element_type=_f32)

  @pl.when(i == GRID - 1)
  def _final():
    outs = []
    for (h0, W0, b0, W1, b1, m1, mh1) in (
        (h0s, W0s, b0s, W1s, b1s, m1s, mh1s),
        (h0d, W0d, b0d, W1d, b1d, m1d, mh1d)):
      x0 = jnp.concatenate([h0[...], mh1[...]], axis=1)
      out0 = jnp.maximum(jnp.dot(x0, W0[...],
                                 preferred_element_type=_f32) + b0[...], 0.0)
      xf = jnp.concatenate([out0, m1[...]], axis=1)
      fin = jnp.dot(xf, W1[...], preferred_element_type=_f32) + b1[...]
      ss = jnp.sum(fin * fin, axis=1, keepdims=True)
      outs.append(fin * lax.rsqrt(jnp.maximum(ss, 1e-12)))
    logits = jnp.sum(outs[0] * outs[1], axis=1, keepdims=True)
    preds[...] = jax.nn.sigmoid(logits)
    lv = lab[...]
    ent = (jnp.maximum(logits, 0.0) - logits * lv
           + jnp.log1p(jnp.exp(-jnp.abs(logits))))
    loss[...] = jnp.mean(ent).reshape(1, 1)


def _tc_dense(h1s, m2s, h1d, m2d, h0s, h0d, lab, agg,
              W0s, b0s, W1s, b1s, W0d, b0d, W1d, b1d):
  blk = pl.BlockSpec((BLK, D), lambda i: (i, 0))
  full = lambda shape: pl.BlockSpec(shape, lambda i: (0, 0))
  return pl.pallas_call(
      _tc_body,
      grid=(GRID,),
      in_specs=[
          blk, blk, blk, blk,
          full((B, D)), full((B, D)), full((B, 1)), full((GRP_BLK, BLK)),
          full((2 * D, 2 * D)), full((1, 2 * D)),
          full((4 * D, 2 * D)), full((1, 2 * D)),
          full((2 * D, 2 * D)), full((1, 2 * D)),
          full((4 * D, 2 * D)), full((1, 2 * D)),
      ],
      out_specs=[full((B, 1)), full((1, 1))],
      out_shape=[
          jax.ShapeDtypeStruct((B, 1), _f32),
          jax.ShapeDtypeStruct((1, 1), _f32),
      ],
      scratch_shapes=[
          pltpu.VMEM((B, 2 * D), _f32),
          pltpu.VMEM((B, D), _f32),
          pltpu.VMEM((B, 2 * D), _f32),
          pltpu.VMEM((B, D), _f32),
      ],
  )(h1s, m2s, h1d, m2d, h0s, h0d, lab, agg,
    W0s, b0s, W1s, b1s, W0d, b0d, W1d, b1d)


def _blockdiag(a, b):
  da, n = a.shape
  db, _ = b.shape
  z = jnp.zeros((da + db, 2 * n), _f32)
  z = z.at[:da, :n].set(a)
  return z.at[da:, n:].set(b)


def kernel(src_graph_nodes, dst_graph_nodes, labels,
           src_nb0, src_nb1, src_nb2, dst_nb0, dst_nb1, dst_nb2,
           node_emb,
           src_Wself0, src_Wneigh0, src_b0, src_Wself1, src_Wneigh1, src_b1,
           dst_Wself0, dst_Wneigh0, dst_b0, dst_Wself1, dst_Wneigh1, dst_b1):
  i32 = jnp.int32
  sgn = src_graph_nodes.astype(i32)
  dgn = dst_graph_nodes.astype(i32)
  snb0 = src_nb0.astype(i32)
  snb1 = src_nb1.astype(i32)
  snb2 = src_nb2.astype(i32)
  dnb0 = dst_nb0.astype(i32)
  dnb1 = dst_nb1.astype(i32)
  dnb2 = dst_nb2.astype(i32)

  s_h0, s_h1, s_m2, d_h0, d_h1, d_m2 = _sc_gather(
      sgn, dgn, node_emb, snb0, snb1, snb2, dnb0, dnb1, dnb2)

  W0s = _blockdiag(src_Wself0, src_Wneigh0)
  W1s = _blockdiag(src_Wself1, src_Wneigh1)
  W0d = _blockdiag(dst_Wself0, dst_Wneigh0)
  W1d = _blockdiag(dst_Wself1, dst_Wneigh1)

  agg = jnp.where(
      (jax.lax.broadcasted_iota(jnp.int32, (GRP_BLK, BLK), 1) // 25)
      == jax.lax.broadcasted_iota(jnp.int32, (GRP_BLK, BLK), 0),
      1.0 / 25.0, 0.0).astype(_f32)

  preds, loss = _tc_dense(
      s_h1, s_m2, d_h1, d_m2, s_h0, d_h0, labels.reshape(B, 1), agg,
      W0s, src_b0.reshape(1, 2 * D), W1s, src_b1.reshape(1, 2 * D),
      W0d, dst_b0.reshape(1, 2 * D), W1d, dst_b1.reshape(1, 2 * D))
  return preds, loss[0, 0]
